# baseline (speedup 1.0000x reference)
import jax, jax.numpy as jnp
from jax.experimental import pallas as pl
from jax.experimental.pallas import tpu as pltpu

G = 100
NPG = 100
NP = 128
N = G * NPG
D = 128
NHID = 128
K1, K1P = 50, 64
K2, K2P = 25, 32


def _eye(n):
    r = jax.lax.broadcasted_iota(jnp.int32, (n, n), 0)
    c = jax.lax.broadcasted_iota(jnp.int32, (n, n), 1)
    return jnp.where(r == c, 1.0, 0.0).astype(jnp.float32)


def _gcn_conv(A, X, W, brow, n, nreal):
    Ah = A + _eye(n)
    dc = jnp.sum(Ah, axis=1, keepdims=True)
    dr = jnp.sum(Ah, axis=0, keepdims=True)
    disc = 1.0 / jnp.sqrt(jnp.clip(dc, 1e-6))
    disr = 1.0 / jnp.sqrt(jnp.clip(dr, 1e-6))
    An = (jnp.broadcast_to(disc, (n, n)) * Ah) * jnp.broadcast_to(disr, (n, n))
    M = jax.lax.dot(An, X, preferred_element_type=jnp.float32)
    Y = jnp.maximum(jax.lax.dot(M, W, preferred_element_type=jnp.float32)
                    + jnp.broadcast_to(brow, (n, NHID)), 0.0)
    ri = jax.lax.broadcasted_iota(jnp.int32, (n, NHID), 0)
    return jnp.where(ri < nreal, Y, 0.0)


def _score_row(A, X, n):
    degc = jnp.clip(jnp.sum(A, axis=1, keepdims=True), 1.0)
    agg = jax.lax.dot(A, X, preferred_element_type=jnp.float32) \
        / jnp.broadcast_to(degc, (n, NHID))
    sc = jnp.sum(jnp.abs(X - agg), axis=1, keepdims=True)
    scT = jnp.transpose(jnp.broadcast_to(sc, (n, NHID)))
    return scT[0:1, :]


def _topk_ranks(scores, n, nreal, k):
    col = jax.lax.broadcasted_iota(jnp.int32, (G, n), 1)
    sc = jnp.where(col < nreal, scores, -1.0)
    rank0 = jnp.full((G, n), 999.0, dtype=jnp.float32)

    def body(j, carry):
        sc, rank = carry
        mx = jnp.max(sc, axis=1, keepdims=True)
        cand = sc == mx
        am = jnp.min(jnp.where(cand, col, n), axis=1, keepdims=True)
        oh = col == am
        rank = jnp.where(oh, j.astype(jnp.float32), rank)
        return jnp.where(oh, -2.0, sc), rank

    _, rank = jax.lax.fori_loop(0, k, body, (sc, rank0))
    return rank


def _P_from_rank(rankrow, kp, n):
    rk = jnp.broadcast_to(rankrow, (kp, n)).astype(jnp.int32)
    rowi = jax.lax.broadcasted_iota(jnp.int32, (kp, n), 0)
    return jnp.where(rk == rowi, 1.0, 0.0).astype(jnp.float32)


def _mega_body(A0_ref, X0_ref, W1_ref, b1_ref, W2_ref, b2_ref, W3_ref, b3_ref,
               X1_ref, X2_ref, X3_ref, sum_ref,
               sc1_ref, rk1_ref, Xp1_ref, A1_ref, sc2_ref, rk2_ref,
               Xp2_ref, A2_ref):
    W1, b1 = W1_ref[...], b1_ref[...]
    W2, b2 = W2_ref[...], b2_ref[...]
    W3, b3 = W3_ref[...], b3_ref[...]
    HI = jax.lax.Precision.HIGHEST

    def stage_a(g, c):
        A = A0_ref[g]
        X1 = _gcn_conv(A, X0_ref[g], W1, b1, NP, NPG)
        X1_ref[g] = X1
        sc1_ref[g] = _score_row(A, X1, NP)
        return c

    jax.lax.fori_loop(0, G, stage_a, 0)
    rk1_ref[...] = _topk_ranks(sc1_ref[...].reshape(G, NP), NP, NPG, K1) \
        .reshape(G, 1, NP)

    def stage_b(g, c):
        P = _P_from_rank(rk1_ref[g], K1P, NP)
        A = A0_ref[g]
        Xp = jax.lax.dot(P, X1_ref[g], precision=HI,
                         preferred_element_type=jnp.float32)
        Xp1_ref[g] = Xp
        Ar = jax.lax.dot(P, A, preferred_element_type=jnp.float32)
        A1 = jax.lax.dot_general(Ar, P, (((1,), (1,)), ((), ())),
                                 preferred_element_type=jnp.float32)
        A1_ref[g] = A1
        X2 = _gcn_conv(A1, Xp, W2, b2, K1P, K1)
        X2_ref[g] = X2
        sc2_ref[g] = _score_row(A1, X2, K1P)
        return c

    jax.lax.fori_loop(0, G, stage_b, 0)
    rk2_ref[...] = _topk_ranks(sc2_ref[...].reshape(G, K1P), K1P, K1, K2) \
        .reshape(G, 1, K1P)

    def stage_c(g, c):
        P = _P_from_rank(rk2_ref[g], K2P, K1P)
        Xp = jax.lax.dot(P, X2_ref[g], precision=HI,
                         preferred_element_type=jnp.float32)
        Xp2_ref[g] = Xp
        Ar = jax.lax.dot(P, A1_ref[g], preferred_element_type=jnp.float32)
        A2 = jax.lax.dot_general(Ar, P, (((1,), (1,)), ((), ())),
                                 preferred_element_type=jnp.float32)
        A2_ref[g] = A2
        X3 = _gcn_conv(A2, Xp, W3, b3, K2P, K2)
        X3_ref[g] = X3

        Xp1 = Xp1_ref[g]
        mx1 = jnp.max(Xp1, axis=0, keepdims=True)
        mn1 = jnp.sum(Xp1, axis=0, keepdims=True) / float(K1)
        mx2 = jnp.max(Xp, axis=0, keepdims=True)
        mn2 = jnp.sum(Xp, axis=0, keepdims=True) / float(K2)
        mx3 = jnp.max(X3, axis=0, keepdims=True)
        mn3 = jnp.sum(X3, axis=0, keepdims=True) / float(K2)
        r = jnp.maximum
        smax = r(mx1, 0.) + r(mx2, 0.) + r(mx3, 0.)
        smean = r(mn1, 0.) + r(mn2, 0.) + r(mn3, 0.)
        sum_ref[g] = jnp.concatenate([smax, smean], axis=1)
        return c

    jax.lax.fori_loop(0, G, stage_c, 0)


def _megakernel(A0p, X0p, W1, b1, W2, b2, W3, b3):
    f32 = jnp.float32
    return pl.pallas_call(
        _mega_body,
        out_shape=(jax.ShapeDtypeStruct((G, NP, NHID), f32),
                   jax.ShapeDtypeStruct((G, K1P, NHID), f32),
                   jax.ShapeDtypeStruct((G, K2P, NHID), f32),
                   jax.ShapeDtypeStruct((G, 1, 2 * NHID), f32)),
        scratch_shapes=[pltpu.VMEM((G, 1, NP), f32),
                        pltpu.VMEM((G, 1, NP), f32),
                        pltpu.VMEM((G, K1P, NHID), f32),
                        pltpu.VMEM((G, K1P, K1P), f32),
                        pltpu.VMEM((G, 1, K1P), f32),
                        pltpu.VMEM((G, 1, K1P), f32),
                        pltpu.VMEM((G, K2P, NHID), f32),
                        pltpu.VMEM((G, K2P, K2P), f32)],
    )(A0p, X0p, W1, b1.reshape(1, NHID), W2, b2.reshape(1, NHID),
      W3, b3.reshape(1, NHID))


def _dense_adj_pad(edge_index):
    ei = edge_index.astype(jnp.int32)
    g = ei[0] // NPG
    s = ei[0] % NPG
    t = ei[1] % NPG
    A = jnp.zeros((G, NP, NP), dtype=jnp.float32).at[g, s, t].add(1.0)
    A = A + jnp.swapaxes(A, 1, 2)
    return A


def kernel(x, edge_index, batch, W1, b1, W2, b2, W3, b3):
    A0p = _dense_adj_pad(edge_index)
    X_pad = jnp.pad(x.reshape(G, NPG, D), ((0, 0), (0, NP - NPG), (0, 0)))

    X1p, X2p, X3p, summary = _megakernel(A0p, X_pad, W1, b1, W2, b2, W3, b3)

    xs0 = X1p[:, :NPG, :].reshape(-1, NHID)
    xs2 = X2p[:, :K1, :].reshape(-1, NHID)
    xs4 = X3p[:, :K2, :].reshape(-1, NHID)
    b0 = batch
    b2_ids = jnp.repeat(jnp.arange(G, dtype=jnp.int32), K1)
    b4_ids = jnp.repeat(jnp.arange(G, dtype=jnp.int32), K2)
    return (summary.reshape(G, 2 * NHID), xs0, xs2, xs4, b0, b2_ids, b4_ids)
